# baseline (speedup 1.0000x reference)
import functools

import jax
import jax.numpy as jnp
from jax import lax
from jax.experimental import pallas as pl
from jax.experimental.pallas import tpu as pltpu
from jax.experimental.pallas import tpu_sc as plsc

NN = 10000
EE = 320000
D = 128
KSTEPS = 10
ALPHA = 0.1

NPAD = 10240
DUMMY = NN
NTILES = 32
CH = 128
NCHUNK = 80
EPAD = NTILES * NCHUNK * CH
RIO = NPAD // 16

BL = 512
GRID = NPAD // BL

_mesh = plsc.VectorSubcoreMesh(core_axis_name="c", subcore_axis_name="s")


@functools.partial(
    pl.kernel,
    mesh=_mesh,
    out_type=jax.ShapeDtypeStruct((2, NPAD, D), jnp.float32),
    scratch_types=[
        pltpu.VMEM((NCHUNK, CH), jnp.int32),
        pltpu.VMEM((CH, D), jnp.float32),
        pltpu.VMEM((CH, D), jnp.float32),
        pltpu.VMEM((CH,), jnp.int32),
        pltpu.VMEM((CH,), jnp.int32),
        pltpu.VMEM_SHARED((NPAD, D), jnp.float32),
        pltpu.SemaphoreType.DMA,
        pltpu.SemaphoreType.DMA,
        pltpu.SemaphoreType.DMA,
        pltpu.SemaphoreType.DMA,
        pltpu.SemaphoreType.DMA,
        pltpu.SemaphoreType.DMA,
    ],
)
def _sc_scatter(g_hbm, ridx_hbm, cidx_hbm, out_hbm, cidx_v, gb0, gb1, rb0, rb1,
                acc, gs0, gs1, is0, is1, ss0, ss1):
    gbufs = (gb0, gb1)
    rbufs = (rb0, rb1)
    gsems = (gs0, gs1)
    isems = (is0, is1)
    ssems = (ss0, ss1)
    c = lax.axis_index("c")
    s = lax.axis_index("s")
    wid = c * 16 + s
    pltpu.sync_copy(cidx_hbm.at[wid], cidx_v)
    zeros16 = jnp.zeros((16,), jnp.float32)

    def _zrow(r, carry):
        for j in range(D // 16):
            gbufs[0][r, pl.ds(j * 16, 16)] = zeros16
        return carry

    lax.fori_loop(0, CH, _zrow, 0)
    base = s * RIO
    for m in range(RIO // CH):
        pltpu.sync_copy(gbufs[0], acc.at[pl.ds(base + m * CH, CH)])
    pltpu.sync_copy(ridx_hbm.at[wid].at[0], rbufs[0])
    pltpu.async_copy(ridx_hbm.at[wid].at[1], rbufs[1], isems[1])
    pltpu.async_copy(g_hbm.at[rbufs[0]], gbufs[0], gsems[0])
    plsc.subcore_barrier()

    def _chunk(j2, carry):
        for b in range(2):
            j = j2 * 2 + b
            o = 1 - b

            @pl.when(j + 1 < NCHUNK)
            def _():
                pltpu.make_async_copy(ridx_hbm.at[wid].at[j + 1], rbufs[o],
                                      isems[o]).wait()

                @pl.when(j >= 1)
                def _():
                    pltpu.make_async_copy(gbufs[o], acc.at[cidx_v.at[j - 1]],
                                          ssems[o]).wait()

                pltpu.async_copy(g_hbm.at[rbufs[o]], gbufs[o], gsems[o])

            pltpu.make_async_copy(g_hbm.at[rbufs[b]], gbufs[b], gsems[b]).wait()

            @pl.when(j + 2 < NCHUNK)
            def _():
                pltpu.async_copy(ridx_hbm.at[wid].at[j + 2], rbufs[b], isems[b])

            pltpu.async_copy(gbufs[b], acc.at[cidx_v.at[j]], ssems[b], add=True)
        return carry

    lax.fori_loop(0, NCHUNK // 2, _chunk, 0)
    for b in range(2):
        j = NCHUNK - 2 + b
        pltpu.make_async_copy(gbufs[b], acc.at[cidx_v.at[j]], ssems[b]).wait()
    plsc.subcore_barrier()
    for m in range(RIO // CH):
        pltpu.sync_copy(acc.at[pl.ds(base + m * CH, CH)], gbufs[0])
        pltpu.sync_copy(gbufs[0], out_hbm.at[c].at[pl.ds(base + m * CH, CH)])


def _mlp_body(x_ref, w1_ref, b1_ref, w2_ref, b2_ref, h_ref, ax_ref):
    h = jnp.dot(x_ref[...], w1_ref[...], preferred_element_type=jnp.float32)
    h = jnp.maximum(h + b1_ref[...], 0.0)
    h = jnp.dot(h, w2_ref[...], preferred_element_type=jnp.float32) + b2_ref[...]
    h_ref[...] = h
    ax_ref[...] = ALPHA * h


def _mlp(xp, w1t, b1r, w2t, b2r):
    return pl.pallas_call(
        _mlp_body,
        grid=(GRID,),
        in_specs=[
            pl.BlockSpec((BL, D), lambda i: (i, 0)),
            pl.BlockSpec((D, D), lambda i: (0, 0)),
            pl.BlockSpec((1, D), lambda i: (0, 0)),
            pl.BlockSpec((D, D), lambda i: (0, 0)),
            pl.BlockSpec((1, D), lambda i: (0, 0)),
        ],
        out_specs=[pl.BlockSpec((BL, D), lambda i: (i, 0))] * 2,
        out_shape=[jax.ShapeDtypeStruct((NPAD, D), jnp.float32)] * 2,
    )(xp, w1t, b1r, w2t, b2r)


def _prep_body(dg_ref, h_ref, dis_ref, g_ref):
    deg = dg_ref[0] + dg_ref[1] + 1.0
    dis = lax.rsqrt(deg)
    dis_ref[...] = dis
    g_ref[...] = dis * h_ref[...]


def _prep(deg2, h0):
    return pl.pallas_call(
        _prep_body,
        grid=(GRID,),
        in_specs=[
            pl.BlockSpec((2, BL, D), lambda i: (0, i, 0)),
            pl.BlockSpec((BL, D), lambda i: (i, 0)),
        ],
        out_specs=[pl.BlockSpec((BL, D), lambda i: (i, 0))] * 2,
        out_shape=[jax.ShapeDtypeStruct((NPAD, D), jnp.float32)] * 2,
    )(deg2, h0)


def _comb_body(acc_ref, g_ref, dis_ref, ax_ref, o_ref, *, final):
    ssum = acc_ref[0] + acc_ref[1] + g_ref[...]
    h = (1.0 - ALPHA) * (dis_ref[...] * ssum) + ax_ref[...]
    o_ref[...] = h if final else dis_ref[...] * h


def _comb(accp, g, dis, ax0, final):
    return pl.pallas_call(
        functools.partial(_comb_body, final=final),
        grid=(GRID,),
        in_specs=[
            pl.BlockSpec((2, BL, D), lambda i: (0, i, 0)),
            pl.BlockSpec((BL, D), lambda i: (i, 0)),
            pl.BlockSpec((BL, D), lambda i: (i, 0)),
            pl.BlockSpec((BL, D), lambda i: (i, 0)),
        ],
        out_specs=pl.BlockSpec((BL, D), lambda i: (i, 0)),
        out_shape=jax.ShapeDtypeStruct((NPAD, D), jnp.float32),
    )(accp, g, dis, ax0)


def kernel(x, edge_index, W1, b1, W2, b2):
    xp = jnp.zeros((NPAD, D), jnp.float32).at[:NN].set(x)
    order = jnp.argsort(edge_index[0])
    row = edge_index[0][order]
    col = edge_index[1][order]
    pad = EPAD - EE
    rowp = jnp.concatenate([row, jnp.zeros((pad,), jnp.int32)]).reshape(
        NTILES, NCHUNK, CH)
    colp = jnp.concatenate([col, jnp.full((pad,), DUMMY, jnp.int32)]).reshape(
        NTILES, NCHUNK, CH)

    h0, ax0 = _mlp(xp, W1.T, b1.reshape(1, D), W2.T, b2.reshape(1, D))
    ones = jnp.ones((NPAD, D), jnp.float32)
    deg2 = _sc_scatter(ones, rowp, colp)
    dis, g = _prep(deg2, h0)
    out = None
    for k in range(KSTEPS):
        accp = _sc_scatter(g, rowp, colp)
        if k + 1 < KSTEPS:
            g = _comb(accp, g, dis, ax0, final=False)
        else:
            out = _comb(accp, g, dis, ax0, final=True)
    return out[:NN]


# speedup vs baseline: 1.0570x; 1.0570x over previous
import functools

import jax
import jax.numpy as jnp
from jax import lax
from jax.experimental import pallas as pl
from jax.experimental.pallas import tpu as pltpu
from jax.experimental.pallas import tpu_sc as plsc

NN = 10000
EE = 320000
D = 128
KSTEPS = 10
ALPHA = 0.1

NPAD = 10240
DUMMY = NN
NTILES = 32
CH = 128
NCHUNK = 80
EPAD = NTILES * NCHUNK * CH
RIO = NPAD // 16

BL = 512
GRID = NPAD // BL

_mesh = plsc.VectorSubcoreMesh(core_axis_name="c", subcore_axis_name="s")


@functools.partial(
    pl.kernel,
    mesh=_mesh,
    out_type=jax.ShapeDtypeStruct((2, NPAD, D), jnp.float32),
    scratch_types=[
        pltpu.VMEM((NCHUNK, CH), jnp.int32),
        pltpu.VMEM((CH, D), jnp.float32),
        pltpu.VMEM((CH, D), jnp.float32),
        pltpu.VMEM((CH,), jnp.int32),
        pltpu.VMEM((CH,), jnp.int32),
        pltpu.VMEM_SHARED((NPAD, D), jnp.float32),
        pltpu.SemaphoreType.DMA,
        pltpu.SemaphoreType.DMA,
        pltpu.SemaphoreType.DMA,
        pltpu.SemaphoreType.DMA,
        pltpu.SemaphoreType.DMA,
        pltpu.SemaphoreType.DMA,
    ],
)
def _sc_scatter(g_hbm, ridx_hbm, cidx_hbm, out_hbm, cidx_v, gb0, gb1, rb0, rb1,
                acc, gs0, gs1, is0, is1, ss0, ss1):
    gbufs = (gb0, gb1)
    rbufs = (rb0, rb1)
    gsems = (gs0, gs1)
    isems = (is0, is1)
    ssems = (ss0, ss1)
    c = lax.axis_index("c")
    s = lax.axis_index("s")
    wid = c * 16 + s
    pltpu.sync_copy(cidx_hbm.at[wid], cidx_v)
    zeros16 = jnp.zeros((16,), jnp.float32)

    def _zrow(r, carry):
        for j in range(D // 16):
            gbufs[0][r, pl.ds(j * 16, 16)] = zeros16
        return carry

    lax.fori_loop(0, CH, _zrow, 0)
    base = s * RIO
    for m in range(RIO // CH):
        pltpu.sync_copy(gbufs[0], acc.at[pl.ds(base + m * CH, CH)])
    pltpu.sync_copy(ridx_hbm.at[wid].at[0], rbufs[0])
    pltpu.async_copy(ridx_hbm.at[wid].at[1], rbufs[1], isems[1])
    pltpu.async_copy(g_hbm.at[rbufs[0]], gbufs[0], gsems[0])
    plsc.subcore_barrier()

    def _chunk(j2, carry):
        for b in range(2):
            j = j2 * 2 + b
            o = 1 - b

            @pl.when(j + 1 < NCHUNK)
            def _():
                pltpu.make_async_copy(ridx_hbm.at[wid].at[j + 1], rbufs[o],
                                      isems[o]).wait()

                @pl.when(j >= 1)
                def _():
                    pltpu.make_async_copy(gbufs[o], acc.at[cidx_v.at[j - 1]],
                                          ssems[o]).wait()

                pltpu.async_copy(g_hbm.at[rbufs[o]], gbufs[o], gsems[o])

            pltpu.make_async_copy(g_hbm.at[rbufs[b]], gbufs[b], gsems[b]).wait()

            @pl.when(j + 2 < NCHUNK)
            def _():
                pltpu.async_copy(ridx_hbm.at[wid].at[j + 2], rbufs[b], isems[b])

            pltpu.async_copy(gbufs[b], acc.at[cidx_v.at[j]], ssems[b], add=True)
        return carry

    lax.fori_loop(0, NCHUNK // 2, _chunk, 0)
    for b in range(2):
        j = NCHUNK - 2 + b
        pltpu.make_async_copy(gbufs[b], acc.at[cidx_v.at[j]], ssems[b]).wait()
    plsc.subcore_barrier()
    for m in range(RIO // CH):
        pltpu.sync_copy(acc.at[pl.ds(base + m * CH, CH)], gbufs[0])
        pltpu.sync_copy(gbufs[0], out_hbm.at[c].at[pl.ds(base + m * CH, CH)])


def _mlp_body(x_ref, w1_ref, b1_ref, w2_ref, b2_ref, h_ref, ax_ref):
    h = jnp.dot(x_ref[...], w1_ref[...], preferred_element_type=jnp.float32)
    h = jnp.maximum(h + b1_ref[...], 0.0)
    h = jnp.dot(h, w2_ref[...], preferred_element_type=jnp.float32) + b2_ref[...]
    h_ref[...] = h
    ax_ref[...] = ALPHA * h


def _mlp(xp, w1t, b1r, w2t, b2r):
    return pl.pallas_call(
        _mlp_body,
        grid=(GRID,),
        in_specs=[
            pl.BlockSpec((BL, D), lambda i: (i, 0)),
            pl.BlockSpec((D, D), lambda i: (0, 0)),
            pl.BlockSpec((1, D), lambda i: (0, 0)),
            pl.BlockSpec((D, D), lambda i: (0, 0)),
            pl.BlockSpec((1, D), lambda i: (0, 0)),
        ],
        out_specs=[pl.BlockSpec((BL, D), lambda i: (i, 0))] * 2,
        out_shape=[jax.ShapeDtypeStruct((NPAD, D), jnp.float32)] * 2,
    )(xp, w1t, b1r, w2t, b2r)


def _prep_body(dg_ref, h_ref, dis_ref, g_ref):
    deg = dg_ref[0] + dg_ref[1] + 1.0
    dis = lax.rsqrt(deg)
    dis_ref[...] = dis
    g_ref[...] = dis * h_ref[...]


def _prep(deg2, h0):
    return pl.pallas_call(
        _prep_body,
        grid=(GRID,),
        in_specs=[
            pl.BlockSpec((2, BL, D), lambda i: (0, i, 0)),
            pl.BlockSpec((BL, D), lambda i: (i, 0)),
        ],
        out_specs=[pl.BlockSpec((BL, D), lambda i: (i, 0))] * 2,
        out_shape=[jax.ShapeDtypeStruct((NPAD, D), jnp.float32)] * 2,
    )(deg2, h0)


def _comb_body(acc_ref, g_ref, dis_ref, ax_ref, o_ref, *, final):
    ssum = acc_ref[0] + acc_ref[1] + g_ref[...]
    h = (1.0 - ALPHA) * (dis_ref[...] * ssum) + ax_ref[...]
    o_ref[...] = h if final else dis_ref[...] * h


def _comb(accp, g, dis, ax0, final):
    return pl.pallas_call(
        functools.partial(_comb_body, final=final),
        grid=(GRID,),
        in_specs=[
            pl.BlockSpec((2, BL, D), lambda i: (0, i, 0)),
            pl.BlockSpec((BL, D), lambda i: (i, 0)),
            pl.BlockSpec((BL, D), lambda i: (i, 0)),
            pl.BlockSpec((BL, D), lambda i: (i, 0)),
        ],
        out_specs=pl.BlockSpec((BL, D), lambda i: (i, 0)),
        out_shape=jax.ShapeDtypeStruct((NPAD, D), jnp.float32),
    )(accp, g, dis, ax0)


def kernel(x, edge_index, W1, b1, W2, b2):
    xp = jnp.zeros((NPAD, D), jnp.float32).at[:NN].set(x)
    row = edge_index[0]
    col = edge_index[1]
    pad = EPAD - EE
    rowp = jnp.concatenate([row, jnp.zeros((pad,), jnp.int32)]).reshape(
        NTILES, NCHUNK, CH)
    colp = jnp.concatenate([col, jnp.full((pad,), DUMMY, jnp.int32)]).reshape(
        NTILES, NCHUNK, CH)

    h0, ax0 = _mlp(xp, W1.T, b1.reshape(1, D), W2.T, b2.reshape(1, D))
    ones = jnp.ones((NPAD, D), jnp.float32)
    deg2 = _sc_scatter(ones, rowp, colp)
    dis, g = _prep(deg2, h0)
    out = None
    for k in range(KSTEPS):
        accp = _sc_scatter(g, rowp, colp)
        if k + 1 < KSTEPS:
            g = _comb(accp, g, dis, ax0, final=False)
        else:
            out = _comb(accp, g, dis, ax0, final=True)
    return out[:NN]


# speedup vs baseline: 1.0720x; 1.0141x over previous
import functools

import jax
import jax.numpy as jnp
from jax import lax
from jax.experimental import pallas as pl
from jax.experimental.pallas import tpu as pltpu
from jax.experimental.pallas import tpu_sc as plsc

NN = 10000
EE = 320000
D = 128
KSTEPS = 10
ALPHA = 0.1

NPAD = 10240
DUMMY = NN
NTILES = 32
CH = 128
NCHUNK = 80
EPAD = NTILES * NCHUNK * CH
RIO = NPAD // 16

BL = 512
GRID = NPAD // BL

_mesh = plsc.VectorSubcoreMesh(core_axis_name="c", subcore_axis_name="s")


@functools.partial(
    pl.kernel,
    mesh=_mesh,
    out_type=jax.ShapeDtypeStruct((2, NPAD, D), jnp.float32),
    scratch_types=[
        pltpu.VMEM((NCHUNK, CH), jnp.int32),
        pltpu.VMEM((CH, D), jnp.float32),
        pltpu.VMEM((CH, D), jnp.float32),
        pltpu.VMEM((CH,), jnp.int32),
        pltpu.VMEM((CH,), jnp.int32),
        pltpu.VMEM_SHARED((NPAD, D), jnp.float32),
        pltpu.SemaphoreType.DMA,
        pltpu.SemaphoreType.DMA,
        pltpu.SemaphoreType.DMA,
        pltpu.SemaphoreType.DMA,
        pltpu.SemaphoreType.DMA,
        pltpu.SemaphoreType.DMA,
    ],
)
def _sc_scatter(g_hbm, ridx_hbm, cidx_hbm, out_hbm, cidx_v, gb0, gb1, rb0, rb1,
                acc, gs0, gs1, is0, is1, ss0, ss1):
    gbufs = (gb0, gb1)
    rbufs = (rb0, rb1)
    gsems = (gs0, gs1)
    isems = (is0, is1)
    ssems = (ss0, ss1)
    c = lax.axis_index("c")
    s = lax.axis_index("s")
    wid = c * 16 + s
    pltpu.sync_copy(cidx_hbm.at[wid], cidx_v)
    zeros16 = jnp.zeros((16,), jnp.float32)

    def _zrow(r, carry):
        for j in range(D // 16):
            gbufs[0][r, pl.ds(j * 16, 16)] = zeros16
        return carry

    lax.fori_loop(0, CH, _zrow, 0)
    base = s * RIO
    for m in range(RIO // CH):
        pltpu.sync_copy(gbufs[0], acc.at[pl.ds(base + m * CH, CH)])
    pltpu.sync_copy(ridx_hbm.at[wid].at[0], rbufs[0])
    pltpu.async_copy(ridx_hbm.at[wid].at[1], rbufs[1], isems[1])
    pltpu.async_copy(g_hbm.at[rbufs[0]], gbufs[0], gsems[0])
    plsc.subcore_barrier()

    def _chunk(j2, carry):
        for b in range(2):
            j = j2 * 2 + b
            o = 1 - b

            @pl.when(j + 1 < NCHUNK)
            def _():
                pltpu.make_async_copy(ridx_hbm.at[wid].at[j + 1], rbufs[o],
                                      isems[o]).wait()

                @pl.when(j >= 1)
                def _():
                    pltpu.make_async_copy(gbufs[o], acc.at[cidx_v.at[j - 1]],
                                          ssems[o]).wait()

                pltpu.async_copy(g_hbm.at[rbufs[o]], gbufs[o], gsems[o])

            pltpu.make_async_copy(g_hbm.at[rbufs[b]], gbufs[b], gsems[b]).wait()

            @pl.when(j + 2 < NCHUNK)
            def _():
                pltpu.async_copy(ridx_hbm.at[wid].at[j + 2], rbufs[b], isems[b])

            pltpu.async_copy(gbufs[b], acc.at[cidx_v.at[j]], ssems[b], add=True)
        return carry

    lax.fori_loop(0, NCHUNK // 2, _chunk, 0)
    for b in range(2):
        j = NCHUNK - 2 + b
        pltpu.make_async_copy(gbufs[b], acc.at[cidx_v.at[j]], ssems[b]).wait()
    plsc.subcore_barrier()
    for m in range(RIO // CH):
        pltpu.sync_copy(acc.at[pl.ds(base + m * CH, CH)], gbufs[0])
        pltpu.sync_copy(gbufs[0], out_hbm.at[c].at[pl.ds(base + m * CH, CH)])


@functools.partial(
    pl.kernel,
    mesh=_mesh,
    out_type=jax.ShapeDtypeStruct((2, NPAD, D), jnp.float32),
    scratch_types=[
        pltpu.VMEM((NCHUNK, CH), jnp.int32),
        pltpu.VMEM((CH, D), jnp.float32),
        pltpu.VMEM_SHARED((NPAD, D), jnp.float32),
        pltpu.SemaphoreType.DMA,
        pltpu.SemaphoreType.DMA,
    ],
)
def _sc_deg(cidx_hbm, out_hbm, cidx_v, gbuf, acc, ss0, ss1):
    ssems = (ss0, ss1)
    c = lax.axis_index("c")
    s = lax.axis_index("s")
    wid = c * 16 + s
    pltpu.sync_copy(cidx_hbm.at[wid], cidx_v)
    zeros16 = jnp.zeros((16,), jnp.float32)
    ones16 = jnp.full((16,), 1.0, jnp.float32)

    def _fill(val):
        def _row(r, carry):
            for j in range(D // 16):
                gbuf[r, pl.ds(j * 16, 16)] = val
            return carry
        lax.fori_loop(0, CH, _row, 0)

    _fill(zeros16)
    base = s * RIO
    for m in range(RIO // CH):
        pltpu.sync_copy(gbuf, acc.at[pl.ds(base + m * CH, CH)])
    _fill(ones16)
    plsc.subcore_barrier()

    def _chunk(j2, carry):
        for b in range(2):
            j = j2 * 2 + b

            @pl.when(j >= 2)
            def _():
                pltpu.make_async_copy(gbuf, acc.at[cidx_v.at[j - 2]],
                                      ssems[b]).wait()

            pltpu.async_copy(gbuf, acc.at[cidx_v.at[j]], ssems[b], add=True)
        return carry

    lax.fori_loop(0, NCHUNK // 2, _chunk, 0)
    for b in range(2):
        j = NCHUNK - 2 + b
        pltpu.make_async_copy(gbuf, acc.at[cidx_v.at[j]], ssems[b]).wait()
    plsc.subcore_barrier()
    for m in range(RIO // CH):
        pltpu.sync_copy(acc.at[pl.ds(base + m * CH, CH)], gbuf)
        pltpu.sync_copy(gbuf, out_hbm.at[c].at[pl.ds(base + m * CH, CH)])


def _mlp_body(x_ref, w1_ref, b1_ref, w2_ref, b2_ref, h_ref, ax_ref):
    h = jnp.dot(x_ref[...], w1_ref[...], preferred_element_type=jnp.float32)
    h = jnp.maximum(h + b1_ref[...], 0.0)
    h = jnp.dot(h, w2_ref[...], preferred_element_type=jnp.float32) + b2_ref[...]
    h_ref[...] = h
    ax_ref[...] = ALPHA * h


def _mlp(xp, w1t, b1r, w2t, b2r):
    return pl.pallas_call(
        _mlp_body,
        grid=(GRID,),
        in_specs=[
            pl.BlockSpec((BL, D), lambda i: (i, 0)),
            pl.BlockSpec((D, D), lambda i: (0, 0)),
            pl.BlockSpec((1, D), lambda i: (0, 0)),
            pl.BlockSpec((D, D), lambda i: (0, 0)),
            pl.BlockSpec((1, D), lambda i: (0, 0)),
        ],
        out_specs=[pl.BlockSpec((BL, D), lambda i: (i, 0))] * 2,
        out_shape=[jax.ShapeDtypeStruct((NPAD, D), jnp.float32)] * 2,
    )(xp, w1t, b1r, w2t, b2r)


def _prep_body(dg_ref, h_ref, dis_ref, g_ref):
    deg = dg_ref[0] + dg_ref[1] + 1.0
    dis = lax.rsqrt(deg)
    dis_ref[...] = dis
    g_ref[...] = dis * h_ref[...]


def _prep(deg2, h0):
    return pl.pallas_call(
        _prep_body,
        grid=(GRID,),
        in_specs=[
            pl.BlockSpec((2, BL, D), lambda i: (0, i, 0)),
            pl.BlockSpec((BL, D), lambda i: (i, 0)),
        ],
        out_specs=[pl.BlockSpec((BL, D), lambda i: (i, 0))] * 2,
        out_shape=[jax.ShapeDtypeStruct((NPAD, D), jnp.float32)] * 2,
    )(deg2, h0)


def _comb_body(acc_ref, g_ref, dis_ref, ax_ref, o_ref, *, final):
    ssum = acc_ref[0] + acc_ref[1] + g_ref[...]
    h = (1.0 - ALPHA) * (dis_ref[...] * ssum) + ax_ref[...]
    o_ref[...] = h if final else dis_ref[...] * h


def _comb(accp, g, dis, ax0, final):
    return pl.pallas_call(
        functools.partial(_comb_body, final=final),
        grid=(GRID,),
        in_specs=[
            pl.BlockSpec((2, BL, D), lambda i: (0, i, 0)),
            pl.BlockSpec((BL, D), lambda i: (i, 0)),
            pl.BlockSpec((BL, D), lambda i: (i, 0)),
            pl.BlockSpec((BL, D), lambda i: (i, 0)),
        ],
        out_specs=pl.BlockSpec((BL, D), lambda i: (i, 0)),
        out_shape=jax.ShapeDtypeStruct((NPAD, D), jnp.float32),
    )(accp, g, dis, ax0)


def kernel(x, edge_index, W1, b1, W2, b2):
    xp = jnp.zeros((NPAD, D), jnp.float32).at[:NN].set(x)
    row = edge_index[0]
    col = edge_index[1]
    pad = EPAD - EE
    rowp = jnp.concatenate([row, jnp.zeros((pad,), jnp.int32)]).reshape(
        NTILES, NCHUNK, CH)
    colp = jnp.concatenate([col, jnp.full((pad,), DUMMY, jnp.int32)]).reshape(
        NTILES, NCHUNK, CH)

    h0, ax0 = _mlp(xp, W1.T, b1.reshape(1, D), W2.T, b2.reshape(1, D))
    deg2 = _sc_deg(colp)
    dis, g = _prep(deg2, h0)
    out = None
    for k in range(KSTEPS):
        accp = _sc_scatter(g, rowp, colp)
        if k + 1 < KSTEPS:
            g = _comb(accp, g, dis, ax0, final=False)
        else:
            out = _comb(accp, g, dis, ax0, final=True)
    return out[:NN]
